# baseline (speedup 1.0000x reference)
import jax
import jax.numpy as jnp
from jax.experimental import pallas as pl
from jax.experimental.pallas import tpu as pltpu

BLOCK_M = 256


def _gnn_block(a_ref, x_ref, wr_ref, wn_ref, b_ref, o_ref):
    i = pl.program_id(1)
    adj = (a_ref[0] != 0).astype(jnp.bfloat16)
    xb = x_ref[0]
    agg = jnp.dot(adj, xb, preferred_element_type=jnp.float32)
    xt = x_ref[0, pl.ds(i * BLOCK_M, BLOCK_M), :]
    acc = jnp.dot(xt, wr_ref[...], preferred_element_type=jnp.float32)
    acc += jnp.dot(agg.astype(jnp.bfloat16), wn_ref[...],
                   preferred_element_type=jnp.float32)
    o_ref[0] = acc + b_ref[0]


def kernel(X, A, W_root, W_nbr, b):
    Bb, N, D = X.shape
    xb = X.astype(jnp.bfloat16)
    wr = W_root.astype(jnp.bfloat16)
    wn = W_nbr.astype(jnp.bfloat16)
    b2 = b.reshape(1, D)
    out = pl.pallas_call(
        _gnn_block,
        grid=(Bb, N // BLOCK_M),
        in_specs=[
            pl.BlockSpec((1, BLOCK_M, N), lambda bb, ii: (bb, ii, 0)),
            pl.BlockSpec((1, N, D), lambda bb, ii: (bb, 0, 0)),
            pl.BlockSpec((D, D), lambda bb, ii: (0, 0)),
            pl.BlockSpec((D, D), lambda bb, ii: (0, 0)),
            pl.BlockSpec((1, D), lambda bb, ii: (0, 0)),
        ],
        out_specs=pl.BlockSpec((1, BLOCK_M, D), lambda bb, ii: (bb, ii, 0)),
        out_shape=jax.ShapeDtypeStruct((Bb, N, D), jnp.float32),
        compiler_params=pltpu.CompilerParams(
            dimension_semantics=("parallel", "arbitrary"),
        ),
    )(A, xb, wr, wn, b2)
    return out


# speedup vs baseline: 1.9763x; 1.9763x over previous
import jax
import jax.numpy as jnp
from jax.experimental import pallas as pl
from jax.experimental.pallas import tpu as pltpu

def _gnn_block(a_ref, x_ref, wr_ref, wn_ref, b_ref, o_ref):
    adj = (a_ref[0] != 0).astype(jnp.bfloat16)
    xb = x_ref[0].astype(jnp.bfloat16)
    agg = jnp.dot(adj, xb, preferred_element_type=jnp.float32)
    acc = jnp.dot(xb, wr_ref[...], preferred_element_type=jnp.float32)
    acc += jnp.dot(agg.astype(jnp.bfloat16), wn_ref[...],
                   preferred_element_type=jnp.float32)
    o_ref[0] = acc + b_ref[0]


def kernel(X, A, W_root, W_nbr, b):
    Bb, N, D = X.shape
    wr = W_root.astype(jnp.bfloat16)
    wn = W_nbr.astype(jnp.bfloat16)
    b2 = b.reshape(1, D)
    out = pl.pallas_call(
        _gnn_block,
        grid=(Bb,),
        in_specs=[
            pl.BlockSpec((1, N, N), lambda bb: (bb, 0, 0)),
            pl.BlockSpec((1, N, D), lambda bb: (bb, 0, 0)),
            pl.BlockSpec((D, D), lambda bb: (0, 0)),
            pl.BlockSpec((D, D), lambda bb: (0, 0)),
            pl.BlockSpec((1, D), lambda bb: (0, 0)),
        ],
        out_specs=pl.BlockSpec((1, N, D), lambda bb: (bb, 0, 0)),
        out_shape=jax.ShapeDtypeStruct((Bb, N, D), jnp.float32),
        compiler_params=pltpu.CompilerParams(
            dimension_semantics=("parallel",),
        ),
    )(A, X, wr, wn, b2)
    return out
